# baseline (device time: 419251 ns/iter reference)
import jax
import jax.numpy as jnp
from jax import lax
from jax.experimental import pallas as pl
from jax.experimental.pallas import tpu as pltpu

CH = 512
NQ = 5
NP = 3
LC = 16


def kernel(x):
    M, N2 = x.shape
    N = N2 // 2
    QR = NQ * CH
    PR = NP * CH
    assert 4 * QR + 4 * PR == M
    ZOFF = 4 * QR
    XOFF = 4 * QR + 2 * PR
    LR = M // LC

    def body(x_ref, out_ref, vbuf, ysend_s, yrecv_s, xf_s, xr_s, zf_s, zr_s,
             xf2_s, xr2_s, zf2_s, zr2_s, xtf_s, xtr_s, ztf_s, ztr_s,
             local_s, dummy_s):
        my_x = lax.axis_index("x")
        my_y = lax.axis_index("y")
        my_z = lax.axis_index("z")
        ypeer = (my_x, 1 - my_y, my_z)
        xpeer = (1 - my_x, my_y, my_z)
        zpeer = (my_x, my_y, 1 - my_z)

        ph = (1 - my_y) * M
        RB = ph + (2 * my_x + my_z) * QR
        XBq = ph + (2 * (1 - my_x) + my_z) * QR
        ZBq = ph + (2 * my_x + (1 - my_z)) * QR
        DB = ph + (2 * (1 - my_x) + (1 - my_z)) * QR
        ZTmine = ph + ZOFF + my_z * PR
        ZTother = ph + ZOFF + (1 - my_z) * PR
        XTmine = ph + XOFF + my_x * PR
        XTother = ph + XOFF + (1 - my_x) * PR

        bar = pltpu.get_barrier_semaphore()
        for p in (ypeer, xpeer, zpeer):
            pl.semaphore_signal(bar, inc=1, device_id=p,
                                device_id_type=pl.DeviceIdType.MESH)
        pl.semaphore_wait(bar, 3)

        pcols = pl.ds((1 - my_y) * N, N)

        def ysend(src_row, dst_row, k):
            r = pltpu.make_async_remote_copy(
                src_ref=x_ref.at[pl.ds(src_row, CH), pcols],
                dst_ref=out_ref.at[pl.ds(dst_row, CH), :],
                send_sem=ysend_s.at[k],
                recv_sem=yrecv_s.at[k],
                device_id=ypeer,
                device_id_type=pl.DeviceIdType.MESH,
            )
            r.start()
            return r

        sends = []
        for c in range(NQ):
            row = (2 * my_x + my_z) * QR + c * CH
            sends.append(ysend(row, my_y * M + row, c))
        for c in range(NP):
            row = ZOFF + my_z * PR + c * CH
            sends.append(ysend(row, my_y * M + row, NQ + c))
        for c in range(NP):
            row = XOFF + my_x * PR + c * CH
            sends.append(ysend(row, my_y * M + row, NQ + NP + c))

        def recv_wait(rows, sem):
            pltpu.make_async_remote_copy(
                src_ref=out_ref.at[rows, :],
                dst_ref=out_ref.at[rows, :],
                send_sem=dummy_s.at[0],
                recv_sem=sem,
                device_id=ypeer,
                device_id_type=pl.DeviceIdType.MESH,
            ).wait_recv()

        def fwd(rows, send_sem, recv_sem, peer):
            r = pltpu.make_async_remote_copy(
                src_ref=out_ref.at[rows, :],
                dst_ref=out_ref.at[rows, :],
                send_sem=send_sem,
                recv_sem=recv_sem,
                device_id=peer,
                device_id_type=pl.DeviceIdType.MESH,
            )
            r.start()
            return r

        local_stores = [None, None]

        def local_chunk(l):
            b = l % 2
            if local_stores[b] is not None:
                local_stores[b].wait()
            ld = pltpu.make_async_copy(
                x_ref.at[pl.ds(l * LR, LR), pl.ds(my_y * N, N)],
                vbuf.at[b], local_s.at[b])
            ld.start()
            ld.wait()
            st = pltpu.make_async_copy(
                vbuf.at[b], out_ref.at[pl.ds(my_y * M + l * LR, LR), :],
                local_s.at[2 + b])
            st.start()
            local_stores[b] = st

        for c in range(NQ):
            local_chunk(2 * c)
            local_chunk(2 * c + 1)
            rows = pl.ds(RB + c * CH, CH)
            recv_wait(rows, yrecv_s.at[c])
            sends.append(fwd(rows, xf_s.at[c], xr_s.at[c], xpeer))
            sends.append(fwd(rows, zf_s.at[c], zr_s.at[c], zpeer))

        for c in range(NP):
            local_chunk(2 * NQ + 2 * c)
            local_chunk(2 * NQ + 2 * c + 1)
            rows = pl.ds(ZTmine + c * CH, CH)
            recv_wait(rows, yrecv_s.at[NQ + c])
            sends.append(fwd(rows, ztf_s.at[c], ztr_s.at[c], zpeer))
            rows = pl.ds(XTmine + c * CH, CH)
            recv_wait(rows, yrecv_s.at[NQ + NP + c])
            sends.append(fwd(rows, xtf_s.at[c], xtr_s.at[c], xpeer))

        for c in range(NQ):
            rows_x = pl.ds(XBq + c * CH, CH)
            recv_wait(rows_x, xr_s.at[c])
            if c % 2 == 0:
                sends.append(fwd(rows_x, zf2_s.at[c // 2], zr2_s.at[c // 2],
                                 zpeer))
            rows_z = pl.ds(ZBq + c * CH, CH)
            recv_wait(rows_z, zr_s.at[c])
            if c % 2 == 1:
                sends.append(fwd(rows_z, xf2_s.at[c // 2], xr2_s.at[c // 2],
                                 xpeer))

        for c in range(NP):
            recv_wait(pl.ds(ZTother + c * CH, CH), ztr_s.at[c])
            recv_wait(pl.ds(XTother + c * CH, CH), xtr_s.at[c])
        for c in range(NQ):
            rows = pl.ds(DB + c * CH, CH)
            if c % 2 == 0:
                recv_wait(rows, zr2_s.at[c // 2])
            else:
                recv_wait(rows, xr2_s.at[c // 2])

        for r in sends:
            r.wait_send()
        local_stores[0].wait()
        local_stores[1].wait()

    NY = NQ + 2 * NP
    return pl.pallas_call(
        body,
        out_shape=jax.ShapeDtypeStruct((2 * M, N), jnp.float32),
        in_specs=[pl.BlockSpec(memory_space=pl.ANY)],
        out_specs=pl.BlockSpec(memory_space=pl.ANY),
        scratch_shapes=[
            pltpu.VMEM((2, M // LC, N), jnp.float32),
            pltpu.SemaphoreType.DMA((NY,)),
            pltpu.SemaphoreType.DMA((NY,)),
            pltpu.SemaphoreType.DMA((NQ,)),
            pltpu.SemaphoreType.DMA((NQ,)),
            pltpu.SemaphoreType.DMA((NQ,)),
            pltpu.SemaphoreType.DMA((NQ,)),
            pltpu.SemaphoreType.DMA((NQ // 2,)),
            pltpu.SemaphoreType.DMA((NQ // 2,)),
            pltpu.SemaphoreType.DMA((NQ // 2 + 1,)),
            pltpu.SemaphoreType.DMA((NQ // 2 + 1,)),
            pltpu.SemaphoreType.DMA((NP,)),
            pltpu.SemaphoreType.DMA((NP,)),
            pltpu.SemaphoreType.DMA((NP,)),
            pltpu.SemaphoreType.DMA((NP,)),
            pltpu.SemaphoreType.DMA((4,)),
            pltpu.SemaphoreType.DMA((1,)),
        ],
        compiler_params=pltpu.CompilerParams(collective_id=0),
    )(x)


# device time: 379392 ns/iter; 1.1051x vs baseline; 1.1051x over previous
import jax
import jax.numpy as jnp
from jax import lax
from jax.experimental import pallas as pl
from jax.experimental.pallas import tpu as pltpu

NC = 16
LC = 16


def kernel(x):
    M, N2 = x.shape
    N = N2 // 2
    QR = M // 4
    CH = QR // NC
    LR = M // LC

    def body(x_ref, out_ref, vbuf, ysend_s, yrecv_s, xf_s, xr_s, zf_s, zr_s,
             xf2_s, xr2_s, zf2_s, zr2_s, local_s, dummy_s):
        my_x = lax.axis_index("x")
        my_y = lax.axis_index("y")
        my_z = lax.axis_index("z")
        ypeer = (my_x, 1 - my_y, my_z)
        xpeer = (1 - my_x, my_y, my_z)
        zpeer = (my_x, my_y, 1 - my_z)

        peer_half = (1 - my_y) * M
        RB = peer_half + (2 * my_x + my_z) * QR
        XB = peer_half + (2 * (1 - my_x) + my_z) * QR
        ZB = peer_half + (2 * my_x + (1 - my_z)) * QR
        DB = peer_half + (2 * (1 - my_x) + (1 - my_z)) * QR

        bar = pltpu.get_barrier_semaphore()
        for p in (ypeer, xpeer, zpeer):
            pl.semaphore_signal(bar, inc=1, device_id=p,
                                device_id_type=pl.DeviceIdType.MESH)
        pl.semaphore_wait(bar, 3)

        ysends = []
        for c in range(NC):
            r = pltpu.make_async_remote_copy(
                src_ref=x_ref.at[pl.ds((2 * my_x + my_z) * QR + c * CH, CH),
                                 pl.ds((1 - my_y) * N, N)],
                dst_ref=out_ref.at[
                    pl.ds(my_y * M + (2 * my_x + my_z) * QR + c * CH, CH), :],
                send_sem=ysend_s.at[c],
                recv_sem=yrecv_s.at[c],
                device_id=ypeer,
                device_id_type=pl.DeviceIdType.MESH,
            )
            r.start()
            ysends.append(r)

        def recv_wait(rows, sem):
            pltpu.make_async_remote_copy(
                src_ref=out_ref.at[rows, :],
                dst_ref=out_ref.at[rows, :],
                send_sem=dummy_s.at[0],
                recv_sem=sem,
                device_id=ypeer,
                device_id_type=pl.DeviceIdType.MESH,
            ).wait_recv()

        def fwd(rows, send_sem, recv_sem, peer):
            r = pltpu.make_async_remote_copy(
                src_ref=out_ref.at[rows, :],
                dst_ref=out_ref.at[rows, :],
                send_sem=send_sem,
                recv_sem=recv_sem,
                device_id=peer,
                device_id_type=pl.DeviceIdType.MESH,
            )
            r.start()
            return r

        local_stores = [None, None]

        def local_chunk(l):
            b = l % 2
            if local_stores[b] is not None:
                local_stores[b].wait()
            ld = pltpu.make_async_copy(
                x_ref.at[pl.ds(l * LR, LR), pl.ds(my_y * N, N)],
                vbuf.at[b], local_s.at[b])
            ld.start()
            ld.wait()
            st = pltpu.make_async_copy(
                vbuf.at[b], out_ref.at[pl.ds(my_y * M + l * LR, LR), :],
                local_s.at[2 + b])
            st.start()
            local_stores[b] = st

        fwds = []
        for c in range(NC):
            if c < LC:
                local_chunk(c)
            rows = pl.ds(RB + c * CH, CH)
            recv_wait(rows, yrecv_s.at[c])
            fwds.append(fwd(rows, xf_s.at[c], xr_s.at[c], xpeer))
            fwds.append(fwd(rows, zf_s.at[c], zr_s.at[c], zpeer))

        for c in range(NC):
            rows_x = pl.ds(XB + c * CH, CH)
            recv_wait(rows_x, xr_s.at[c])
            if c % 2 == 0:
                fwds.append(fwd(rows_x, zf2_s.at[c // 2], zr2_s.at[c // 2],
                                zpeer))
            rows_z = pl.ds(ZB + c * CH, CH)
            recv_wait(rows_z, zr_s.at[c])
            if c % 2 == 1:
                fwds.append(fwd(rows_z, xf2_s.at[c // 2], xr2_s.at[c // 2],
                                xpeer))

        for c in range(NC):
            rows = pl.ds(DB + c * CH, CH)
            if c % 2 == 0:
                recv_wait(rows, zr2_s.at[c // 2])
            else:
                recv_wait(rows, xr2_s.at[c // 2])

        for r in ysends:
            r.wait_send()
        for r in fwds:
            r.wait_send()
        local_stores[0].wait()
        local_stores[1].wait()

    return pl.pallas_call(
        body,
        out_shape=jax.ShapeDtypeStruct((2 * M, N), jnp.float32),
        in_specs=[pl.BlockSpec(memory_space=pl.ANY)],
        out_specs=pl.BlockSpec(memory_space=pl.ANY),
        scratch_shapes=[
            pltpu.VMEM((2, M // LC, N), jnp.float32),
            pltpu.SemaphoreType.DMA((NC,)),
            pltpu.SemaphoreType.DMA((NC,)),
            pltpu.SemaphoreType.DMA((NC,)),
            pltpu.SemaphoreType.DMA((NC,)),
            pltpu.SemaphoreType.DMA((NC,)),
            pltpu.SemaphoreType.DMA((NC,)),
            pltpu.SemaphoreType.DMA((NC // 2,)),
            pltpu.SemaphoreType.DMA((NC // 2,)),
            pltpu.SemaphoreType.DMA((NC // 2,)),
            pltpu.SemaphoreType.DMA((NC // 2,)),
            pltpu.SemaphoreType.DMA((4,)),
            pltpu.SemaphoreType.DMA((1,)),
        ],
        compiler_params=pltpu.CompilerParams(collective_id=0),
    )(x)
